# baseline (speedup 1.0000x reference)
import functools

import jax
import jax.numpy as jnp
from jax import lax
from jax.experimental import pallas as pl
from jax.experimental.pallas import tpu as pltpu
from jax.experimental.pallas import tpu_sc as plsc

N = 10000
E = 320000
D = 128
NC = 2
NS = 16
NW = NC * NS
CHUNK = 128
EPAD = 327680
CPW = EPAD // (NW * CHUNK)
NPAD = 10240
RPT = NPAD // NS
CW = 16
ZR = 8


def _mesh():
    return plsc.VectorSubcoreMesh(core_axis_name="c", subcore_axis_name="s",
                                  num_cores=NC, num_subcores=NS)


def _sc_sum_body(with_counts, *refs):
    if with_counts:
        (table, packed3, out_p0, out_p1, out_c,
         packed_v, srcl, dstl, rows0, rows1, rows2, frow,
         g0, g1, g2, s0, s1, s2, c0, c1, c2, ones_v, zc, accum, cnt) = refs
        csem = (c0, c1, c2)
    else:
        (table, packed3, out_p0, out_p1,
         packed_v, srcl, dstl, rows0, rows1, rows2, frow,
         g0, g1, g2, s0, s1, s2, accum) = refs
    cid = lax.axis_index("c")
    sid = lax.axis_index("s")
    wid = sid * NC + cid

    rows = (rows0, rows1, rows2)
    gsem = (g0, g1, g2)
    ssem = (s0, s1, s2)

    zero32 = jnp.zeros((32,), jnp.bfloat16)

    def zrow(r, c):
        for c8 in range(D // 32):
            rows0[r, pl.ds(c8 * 32, 32)] = zero32
        return c
    lax.fori_loop(0, CHUNK, zrow, 0)

    def zcopy(k, c):
        pltpu.sync_copy(rows0, accum.at[pl.ds(sid * RPT + k * CHUNK, CHUNK), :])
        return c
    lax.fori_loop(0, RPT // CHUNK, zcopy, 0)

    if with_counts:
        zero16 = jnp.zeros((16,), jnp.float32)
        one16 = jnp.ones((16,), jnp.float32)

        def zcrow(r, c):
            zc[r, :] = zero16
            return c
        lax.fori_loop(0, RPT, zcrow, 0)
        pltpu.sync_copy(zc, cnt.at[pl.ds(sid * RPT, RPT), :])

        def orow(r, c):
            ones_v[r, :] = one16
            return c
        lax.fori_loop(0, CHUNK, orow, 0)

    plsc.subcore_barrier()

    pltpu.sync_copy(packed3.at[wid], packed_v)

    mask14 = jnp.full((16,), 0x3FFF, jnp.int32)

    def unpack(j, b):
        for k in range(CHUNK // 16):
            v = packed_v[j, pl.ds(k * 16, 16)]
            srcl[b, pl.ds(k * 16, 16)] = v & mask14
            dstl[b, pl.ds(k * 16, 16)] = lax.shift_right_logical(
                v, jnp.full((16,), 14, jnp.int32)) & mask14

    def sg(b, j):
        pltpu.async_copy(table.at[srcl.at[b]], rows[b], gsem[b])

    def wg(b):
        pltpu.make_async_copy(table.at[pl.ds(0, CHUNK)], rows[b],
                              gsem[b]).wait()

    def ss(b):
        pltpu.async_copy(rows[b], accum.at[dstl.at[b]], ssem[b], add=True)
        if with_counts:
            pltpu.async_copy(ones_v, cnt.at[dstl.at[b]], csem[b], add=True)

    def ws(b):
        pltpu.make_async_copy(table.at[pl.ds(0, CHUNK)], rows[b],
                              ssem[b]).wait()
        if with_counts:
            pltpu.make_async_copy(out_c.at[0, pl.ds(0, CHUNK), :], ones_v,
                                  csem[b]).wait()

    unpack(0, 0); sg(0, 0)
    unpack(1, 1); sg(1, 1)
    wg(0); ss(0)
    unpack(2, 2); sg(2, 2)
    wg(1); ss(1)
    ws(0); unpack(3, 0); sg(0, 3)
    wg(2); ss(2)
    ws(1); unpack(4, 1); sg(1, 4)

    def tri(t, c):
        j0 = 3 * t
        wg(0); ss(0)
        ws(2); unpack(j0 + 2, 2); sg(2, j0 + 2)
        wg(1); ss(1)
        ws(0); unpack(j0 + 3, 0); sg(0, j0 + 3)
        wg(2); ss(2)
        ws(1); unpack(j0 + 4, 1); sg(1, j0 + 4)
        return c
    lax.fori_loop(1, (CPW - 5) // 3 + 1, tri, 0)

    wg(0); ss(0)
    wg(1); ws(2); ss(1)
    ws(0); ws(1)

    plsc.subcore_barrier()

    sh16 = jnp.full((16,), 16, jnp.int32)
    himask = jnp.full((16,), -65536, jnp.int32)

    def conv_chunk(k, c):
        base = sid * RPT + k * CHUNK
        pltpu.sync_copy(accum.at[pl.ds(base, CHUNK), :], rows0)

        def conv_row(r, c2):
            for g in range(D // 32):
                w = plsc.bitcast(rows0[r, pl.ds(32 * g, 32)], jnp.int32)
                ev = plsc.bitcast(lax.shift_left(w, sh16), jnp.float32)
                od = plsc.bitcast(w & himask, jnp.float32)
                frow[r, pl.ds(16 * g, 16)] = ev
                frow[r, pl.ds(D // 2 + 16 * g, 16)] = od
            return c2
        lax.fori_loop(0, CHUNK, conv_row, 0)

        @pl.when(cid == 0)
        def _():
            pltpu.sync_copy(frow, out_p0.at[pl.ds(base, CHUNK), :])

        @pl.when(cid == 1)
        def _():
            pltpu.sync_copy(frow, out_p1.at[pl.ds(base, CHUNK), :])
        return c
    lax.fori_loop(0, RPT // CHUNK, conv_chunk, 0)

    if with_counts:
        pltpu.sync_copy(cnt.at[pl.ds(sid * RPT, RPT), :],
                        out_c.at[cid, pl.ds(sid * RPT, RPT), :])


def _make_sc_sum(with_counts):
    out_type = [jax.ShapeDtypeStruct((NPAD, D), jnp.float32),
                jax.ShapeDtypeStruct((NPAD, D), jnp.float32)]
    scratch = [
        pltpu.VMEM((CPW, CHUNK), jnp.int32),
        pltpu.VMEM((3, CHUNK), jnp.int32),
        pltpu.VMEM((3, CHUNK), jnp.int32),
        pltpu.VMEM((CHUNK, D), jnp.bfloat16),
        pltpu.VMEM((CHUNK, D), jnp.bfloat16),
        pltpu.VMEM((CHUNK, D), jnp.bfloat16),
        pltpu.VMEM((CHUNK, D), jnp.float32),
        pltpu.SemaphoreType.DMA,
        pltpu.SemaphoreType.DMA,
        pltpu.SemaphoreType.DMA,
        pltpu.SemaphoreType.DMA,
        pltpu.SemaphoreType.DMA,
        pltpu.SemaphoreType.DMA,
    ]
    if with_counts:
        out_type.append(jax.ShapeDtypeStruct((NC, NPAD, CW), jnp.float32))
        scratch += [
            pltpu.SemaphoreType.DMA,
            pltpu.SemaphoreType.DMA,
            pltpu.SemaphoreType.DMA,
            pltpu.VMEM((CHUNK, CW), jnp.float32),
            pltpu.VMEM((RPT, CW), jnp.float32),
        ]
    scratch.append(pltpu.VMEM_SHARED((NPAD, D), jnp.bfloat16))
    if with_counts:
        scratch.append(pltpu.VMEM_SHARED((NPAD, CW), jnp.float32))
    return pl.kernel(
        functools.partial(_sc_sum_body, with_counts),
        out_type=tuple(out_type),
        mesh=_mesh(),
        scratch_types=tuple(scratch),
        compiler_params=pltpu.CompilerParams(use_tc_tiling_on_sc=False,
                                             needs_layout_passes=False),
        name="sc_segment_sum" + ("_counts" if with_counts else ""),
    )


_BN = 1000


def _pack_body(e_ref, pk_ref):
    pk_ref[pl.ds(0, E)] = e_ref[0] | (e_ref[1] << 14)
    pad_dst = N + lax.rem(lax.iota(jnp.int32, EPAD - E), jnp.int32(NPAD - N))
    pk_ref[pl.ds(E, EPAD - E)] = pad_dst << 14


def _pack(e):
    return pl.pallas_call(
        _pack_body,
        out_shape=jax.ShapeDtypeStruct((EPAD,), jnp.int32),
    )(e)


def _proj_body(x_ref, w_ref, b_ref, o_ref, ob_ref):
    t = lax.dot_general(
        x_ref[...], w_ref[...], (((1,), (1,)), ((), ())),
        preferred_element_type=jnp.float32) + b_ref[...]
    o_ref[...] = t
    ob_ref[...] = t.astype(jnp.bfloat16).reshape(_BN * D)


def _proj(x, w, b):
    return pl.pallas_call(
        _proj_body,
        grid=(N // _BN,),
        in_specs=[
            pl.BlockSpec((_BN, D), lambda i: (i, 0)),
            pl.BlockSpec((D, D), lambda i: (0, 0)),
            pl.BlockSpec((1, D), lambda i: (0, 0)),
        ],
        out_specs=[pl.BlockSpec((_BN, D), lambda i: (i, 0)),
                   pl.BlockSpec((_BN * D,), lambda i: (i,))],
        out_shape=[jax.ShapeDtypeStruct((N, D), jnp.float32),
                   jax.ShapeDtypeStruct((N * D,), jnp.bfloat16)],
    )(x, w, b.reshape(1, D))


def _layer1_body(p0_ref, p1_ref, c_ref, h_ref, wl_ref, bl_ref, wr_ref,
                 o_ref, ob_ref, inv_ref):
    summed = p0_ref[...] + p1_ref[...]
    inv = 1.0 / jnp.maximum(c_ref[0] + c_ref[1], 1.0)
    inv_ref[...] = inv
    t = (lax.dot_general(summed * inv[:, 0:1], wl_ref[...],
                         (((1,), (1,)), ((), ())),
                         preferred_element_type=jnp.float32)
         + bl_ref[...]
         + lax.dot_general(h_ref[...], wr_ref[...], (((1,), (1,)), ((), ())),
                           preferred_element_type=jnp.float32))
    t = jnp.maximum(t, 0.0) + h_ref[...]
    o_ref[...] = t
    ob_ref[...] = t.astype(jnp.bfloat16).reshape(_BN * D)


def _layer1(p0, p1, c, h, wl, bl, wr):
    return pl.pallas_call(
        _layer1_body,
        grid=(N // _BN,),
        in_specs=[
            pl.BlockSpec((_BN, D), lambda i: (i, 0)),
            pl.BlockSpec((_BN, D), lambda i: (i, 0)),
            pl.BlockSpec((NC, _BN, CW), lambda i: (0, i, 0)),
            pl.BlockSpec((_BN, D), lambda i: (i, 0)),
            pl.BlockSpec((D, D), lambda i: (0, 0)),
            pl.BlockSpec((1, D), lambda i: (0, 0)),
            pl.BlockSpec((D, D), lambda i: (0, 0)),
        ],
        out_specs=[pl.BlockSpec((_BN, D), lambda i: (i, 0)),
                   pl.BlockSpec((_BN * D,), lambda i: (i,)),
                   pl.BlockSpec((_BN, CW), lambda i: (i, 0))],
        out_shape=[jax.ShapeDtypeStruct((N, D), jnp.float32),
                   jax.ShapeDtypeStruct((N * D,), jnp.bfloat16),
                   jax.ShapeDtypeStruct((N, CW), jnp.float32)],
    )(p0, p1, c, h, wl, bl.reshape(1, D), wr)


def _layer2_body(p0_ref, p1_ref, inv_ref, h_ref, wl_ref, bl_ref, wr_ref,
                 o_ref):
    mean = (p0_ref[...] + p1_ref[...]) * inv_ref[:, 0:1]
    t = (lax.dot_general(mean, wl_ref[...], (((1,), (1,)), ((), ())),
                         preferred_element_type=jnp.float32)
         + bl_ref[...]
         + lax.dot_general(h_ref[...], wr_ref[...], (((1,), (1,)), ((), ())),
                           preferred_element_type=jnp.float32))
    o_ref[...] = t + h_ref[...]


def _layer2(p0, p1, inv, h, wl, bl, wr):
    return pl.pallas_call(
        _layer2_body,
        grid=(N // _BN,),
        in_specs=[
            pl.BlockSpec((_BN, D), lambda i: (i, 0)),
            pl.BlockSpec((_BN, D), lambda i: (i, 0)),
            pl.BlockSpec((_BN, CW), lambda i: (i, 0)),
            pl.BlockSpec((_BN, D), lambda i: (i, 0)),
            pl.BlockSpec((D, D), lambda i: (0, 0)),
            pl.BlockSpec((1, D), lambda i: (0, 0)),
            pl.BlockSpec((D, D), lambda i: (0, 0)),
        ],
        out_specs=pl.BlockSpec((_BN, D), lambda i: (i, 0)),
        out_shape=jax.ShapeDtypeStruct((N, D), jnp.float32),
    )(p0, p1, inv, h, wl, bl.reshape(1, D), wr)


_sc_sum_counts = _make_sc_sum(True)
_sc_sum_plain = _make_sc_sum(False)


def kernel(x, edge_index, W_lin, b_lin, W_l1, b_l1, W_r1, W_l2, b_l2, W_r2):
    perm = jnp.concatenate([jnp.arange(0, D, 2), jnp.arange(1, D, 2)])
    W_l1p = W_l1[:, perm]
    W_l2p = W_l2[:, perm]
    packed3 = _pack(edge_index).reshape(NW, CPW, CHUNK)
    h, h_bf = _proj(x, W_lin, b_lin)
    p0, p1, c = _sc_sum_counts(h_bf.reshape(N, D), packed3)
    out1, out1_bf, inv = _layer1(p0, p1, c, h, W_l1p, b_l1, W_r1)
    q0, q1 = _sc_sum_plain(out1_bf.reshape(N, D), packed3)
    out2 = _layer2(q0, q1, inv, out1, W_l2p, b_l2, W_r2)
    return (out1, out2)


# speedup vs baseline: 2.3254x; 2.3254x over previous
import functools

import jax
import jax.numpy as jnp
from jax import lax
from jax.experimental import pallas as pl
from jax.experimental.pallas import tpu as pltpu
from jax.experimental.pallas import tpu_sc as plsc

N = 10000
E = 320000
D = 128
NC = 2
NS = 16
NW = NC * NS
CHUNK = 80
EPAD = E
CPW = EPAD // (NW * CHUNK)
NPAD = 10240
RPT = NPAD // NS
CW = 16
ZR = 8


def _mesh():
    return plsc.VectorSubcoreMesh(core_axis_name="c", subcore_axis_name="s",
                                  num_cores=NC, num_subcores=NS)


def _sc_sum_body(with_counts, *refs):
    if with_counts:
        (table, packed3, out_p0, out_p1, out_c,
         packed_v, srcl, dstl, rows0, rows1, rows2, frow,
         g0, g1, g2, s0, s1, s2, c0, c1, c2, ones_v, zc, accum, cnt) = refs
        csem = (c0, c1, c2)
    else:
        (table, packed3, out_p0, out_p1,
         packed_v, srcl, dstl, rows0, rows1, rows2, frow,
         g0, g1, g2, s0, s1, s2, accum) = refs
    cid = lax.axis_index("c")
    sid = lax.axis_index("s")
    wid = sid * NC + cid

    rows = (rows0, rows1, rows2)
    gsem = (g0, g1, g2)
    ssem = (s0, s1, s2)

    zero32 = jnp.zeros((32,), jnp.bfloat16)

    def zrow(r, c):
        for c8 in range(D // 32):
            rows0[r, pl.ds(c8 * 32, 32)] = zero32
        return c
    lax.fori_loop(0, CHUNK, zrow, 0)

    def zcopy(k, c):
        pltpu.sync_copy(rows0, accum.at[pl.ds(sid * RPT + k * CHUNK, CHUNK), :])
        return c
    lax.fori_loop(0, RPT // CHUNK, zcopy, 0)

    if with_counts:
        zero16 = jnp.zeros((16,), jnp.float32)
        one16 = jnp.ones((16,), jnp.float32)

        def zcrow(r, c):
            zc[r, :] = zero16
            return c
        lax.fori_loop(0, RPT, zcrow, 0)
        pltpu.sync_copy(zc, cnt.at[pl.ds(sid * RPT, RPT), :])

        def orow(r, c):
            ones_v[r, :] = one16
            return c
        lax.fori_loop(0, CHUNK, orow, 0)

    plsc.subcore_barrier()

    pltpu.sync_copy(packed3.at[wid], packed_v)

    mask14 = jnp.full((16,), 0x3FFF, jnp.int32)

    def unpack(j, b):
        for k in range(CHUNK // 16):
            v = packed_v[j, pl.ds(k * 16, 16)]
            srcl[b, pl.ds(k * 16, 16)] = v & mask14
            dstl[b, pl.ds(k * 16, 16)] = lax.shift_right_logical(
                v, jnp.full((16,), 14, jnp.int32)) & mask14

    def sg(b, j):
        pltpu.async_copy(table.at[srcl.at[b]], rows[b], gsem[b])

    def wg(b):
        pltpu.make_async_copy(table.at[pl.ds(0, CHUNK)], rows[b],
                              gsem[b]).wait()

    def ss(b):
        pltpu.async_copy(rows[b], accum.at[dstl.at[b]], ssem[b], add=True)
        if with_counts:
            pltpu.async_copy(ones_v, cnt.at[dstl.at[b]], csem[b], add=True)

    def ws(b):
        pltpu.make_async_copy(table.at[pl.ds(0, CHUNK)], rows[b],
                              ssem[b]).wait()
        if with_counts:
            pltpu.make_async_copy(out_c.at[0, pl.ds(0, CHUNK), :], ones_v,
                                  csem[b]).wait()

    unpack(0, 0); sg(0, 0)
    unpack(1, 1); sg(1, 1)
    wg(0); ss(0)
    unpack(2, 2); sg(2, 2)
    wg(1); ss(1)
    ws(0); unpack(3, 0); sg(0, 3)
    wg(2); ss(2)
    ws(1); unpack(4, 1); sg(1, 4)

    def tri(t, c):
        j0 = 3 * t
        wg(0); ss(0)
        ws(2); unpack(j0 + 2, 2); sg(2, j0 + 2)
        wg(1); ss(1)
        ws(0); unpack(j0 + 3, 0); sg(0, j0 + 3)
        wg(2); ss(2)
        ws(1); unpack(j0 + 4, 1); sg(1, j0 + 4)
        return c
    lax.fori_loop(1, (CPW - 5) // 3 + 1, tri, 0)

    wg(0); ss(0)
    wg(1); ws(2); ss(1)
    ws(0); ws(1)

    plsc.subcore_barrier()

    sh16 = jnp.full((16,), 16, jnp.int32)
    himask = jnp.full((16,), -65536, jnp.int32)

    def conv_chunk(k, c):
        base = sid * RPT + k * CHUNK
        pltpu.sync_copy(accum.at[pl.ds(base, CHUNK), :], rows0)

        def conv_row(r, c2):
            for g in range(D // 32):
                w = plsc.bitcast(rows0[r, pl.ds(32 * g, 32)], jnp.int32)
                ev = plsc.bitcast(lax.shift_left(w, sh16), jnp.float32)
                od = plsc.bitcast(w & himask, jnp.float32)
                frow[r, pl.ds(16 * g, 16)] = ev
                frow[r, pl.ds(D // 2 + 16 * g, 16)] = od
            return c2
        lax.fori_loop(0, CHUNK, conv_row, 0)

        @pl.when(cid == 0)
        def _():
            pltpu.sync_copy(frow, out_p0.at[pl.ds(base, CHUNK), :])

        @pl.when(cid == 1)
        def _():
            pltpu.sync_copy(frow, out_p1.at[pl.ds(base, CHUNK), :])
        return c
    lax.fori_loop(0, RPT // CHUNK, conv_chunk, 0)

    if with_counts:
        pltpu.sync_copy(cnt.at[pl.ds(sid * RPT, RPT), :],
                        out_c.at[cid, pl.ds(sid * RPT, RPT), :])


def _make_sc_sum(with_counts):
    out_type = [jax.ShapeDtypeStruct((NPAD, D), jnp.float32),
                jax.ShapeDtypeStruct((NPAD, D), jnp.float32)]
    scratch = [
        pltpu.VMEM((CPW, CHUNK), jnp.int32),
        pltpu.VMEM((3, CHUNK), jnp.int32),
        pltpu.VMEM((3, CHUNK), jnp.int32),
        pltpu.VMEM((CHUNK, D), jnp.bfloat16),
        pltpu.VMEM((CHUNK, D), jnp.bfloat16),
        pltpu.VMEM((CHUNK, D), jnp.bfloat16),
        pltpu.VMEM((CHUNK, D), jnp.float32),
        pltpu.SemaphoreType.DMA,
        pltpu.SemaphoreType.DMA,
        pltpu.SemaphoreType.DMA,
        pltpu.SemaphoreType.DMA,
        pltpu.SemaphoreType.DMA,
        pltpu.SemaphoreType.DMA,
    ]
    if with_counts:
        out_type.append(jax.ShapeDtypeStruct((NC, NPAD, CW), jnp.float32))
        scratch += [
            pltpu.SemaphoreType.DMA,
            pltpu.SemaphoreType.DMA,
            pltpu.SemaphoreType.DMA,
            pltpu.VMEM((CHUNK, CW), jnp.float32),
            pltpu.VMEM((RPT, CW), jnp.float32),
        ]
    scratch.append(pltpu.VMEM_SHARED((NPAD, D), jnp.bfloat16))
    if with_counts:
        scratch.append(pltpu.VMEM_SHARED((NPAD, CW), jnp.float32))
    return pl.kernel(
        functools.partial(_sc_sum_body, with_counts),
        out_type=tuple(out_type),
        mesh=_mesh(),
        scratch_types=tuple(scratch),
        compiler_params=pltpu.CompilerParams(use_tc_tiling_on_sc=False,
                                             needs_layout_passes=False),
        name="sc_segment_sum" + ("_counts" if with_counts else ""),
    )


_BN = 1000


def _pack_body(e_ref, pk_ref):
    pk_ref[...] = e_ref[0] | (e_ref[1] << 14)


def _pack(e):
    return pl.pallas_call(
        _pack_body,
        out_shape=jax.ShapeDtypeStruct((EPAD,), jnp.int32),
    )(e)


def _proj_body(x_ref, w_ref, b_ref, o_ref, ob_ref):
    t = lax.dot_general(
        x_ref[...], w_ref[...], (((1,), (1,)), ((), ())),
        preferred_element_type=jnp.float32) + b_ref[...]
    o_ref[...] = t
    ob_ref[...] = t.astype(jnp.bfloat16).reshape(_BN * D)


def _proj(x, w, b):
    return pl.pallas_call(
        _proj_body,
        grid=(N // _BN,),
        in_specs=[
            pl.BlockSpec((_BN, D), lambda i: (i, 0)),
            pl.BlockSpec((D, D), lambda i: (0, 0)),
            pl.BlockSpec((1, D), lambda i: (0, 0)),
        ],
        out_specs=[pl.BlockSpec((_BN, D), lambda i: (i, 0)),
                   pl.BlockSpec((_BN * D,), lambda i: (i,))],
        out_shape=[jax.ShapeDtypeStruct((N, D), jnp.float32),
                   jax.ShapeDtypeStruct((N * D,), jnp.bfloat16)],
    )(x, w, b.reshape(1, D))


def _layer1_body(p0_ref, p1_ref, c_ref, h_ref, wl_ref, bl_ref, wr_ref,
                 o_ref, ob_ref, inv_ref):
    summed = p0_ref[...] + p1_ref[...]
    inv = 1.0 / jnp.maximum(c_ref[0] + c_ref[1], 1.0)
    inv_ref[...] = inv
    t = (lax.dot_general(summed * inv[:, 0:1], wl_ref[...],
                         (((1,), (1,)), ((), ())),
                         preferred_element_type=jnp.float32)
         + bl_ref[...]
         + lax.dot_general(h_ref[...], wr_ref[...], (((1,), (1,)), ((), ())),
                           preferred_element_type=jnp.float32))
    t = jnp.maximum(t, 0.0) + h_ref[...]
    o_ref[...] = t
    ob_ref[...] = t.astype(jnp.bfloat16).reshape(_BN * D)


def _layer1(p0, p1, c, h, wl, bl, wr):
    return pl.pallas_call(
        _layer1_body,
        grid=(N // _BN,),
        in_specs=[
            pl.BlockSpec((_BN, D), lambda i: (i, 0)),
            pl.BlockSpec((_BN, D), lambda i: (i, 0)),
            pl.BlockSpec((NC, _BN, CW), lambda i: (0, i, 0)),
            pl.BlockSpec((_BN, D), lambda i: (i, 0)),
            pl.BlockSpec((D, D), lambda i: (0, 0)),
            pl.BlockSpec((1, D), lambda i: (0, 0)),
            pl.BlockSpec((D, D), lambda i: (0, 0)),
        ],
        out_specs=[pl.BlockSpec((_BN, D), lambda i: (i, 0)),
                   pl.BlockSpec((_BN * D,), lambda i: (i,)),
                   pl.BlockSpec((_BN, CW), lambda i: (i, 0))],
        out_shape=[jax.ShapeDtypeStruct((N, D), jnp.float32),
                   jax.ShapeDtypeStruct((N * D,), jnp.bfloat16),
                   jax.ShapeDtypeStruct((N, CW), jnp.float32)],
    )(p0, p1, c, h, wl, bl.reshape(1, D), wr)


def _layer2_body(p0_ref, p1_ref, inv_ref, h_ref, wl_ref, bl_ref, wr_ref,
                 o_ref):
    mean = (p0_ref[...] + p1_ref[...]) * inv_ref[:, 0:1]
    t = (lax.dot_general(mean, wl_ref[...], (((1,), (1,)), ((), ())),
                         preferred_element_type=jnp.float32)
         + bl_ref[...]
         + lax.dot_general(h_ref[...], wr_ref[...], (((1,), (1,)), ((), ())),
                           preferred_element_type=jnp.float32))
    o_ref[...] = t + h_ref[...]


def _layer2(p0, p1, inv, h, wl, bl, wr):
    return pl.pallas_call(
        _layer2_body,
        grid=(N // _BN,),
        in_specs=[
            pl.BlockSpec((_BN, D), lambda i: (i, 0)),
            pl.BlockSpec((_BN, D), lambda i: (i, 0)),
            pl.BlockSpec((_BN, CW), lambda i: (i, 0)),
            pl.BlockSpec((_BN, D), lambda i: (i, 0)),
            pl.BlockSpec((D, D), lambda i: (0, 0)),
            pl.BlockSpec((1, D), lambda i: (0, 0)),
            pl.BlockSpec((D, D), lambda i: (0, 0)),
        ],
        out_specs=pl.BlockSpec((_BN, D), lambda i: (i, 0)),
        out_shape=jax.ShapeDtypeStruct((N, D), jnp.float32),
    )(p0, p1, inv, h, wl, bl.reshape(1, D), wr)


_sc_sum_counts = _make_sc_sum(True)
_sc_sum_plain = _make_sc_sum(False)


def kernel(x, edge_index, W_lin, b_lin, W_l1, b_l1, W_r1, W_l2, b_l2, W_r2):
    perm = jnp.concatenate([jnp.arange(0, D, 2), jnp.arange(1, D, 2)])
    W_l1p = W_l1[:, perm]
    W_l2p = W_l2[:, perm]
    packed3 = _pack(edge_index).reshape(NW, CPW, CHUNK)
    h, h_bf = _proj(x, W_lin, b_lin)
    p0, p1, c = _sc_sum_counts(h_bf.reshape(N, D), packed3)
    out1, out1_bf, inv = _layer1(p0, p1, c, h, W_l1p, b_l1, W_r1)
    q0, q1 = _sc_sum_plain(out1_bf.reshape(N, D), packed3)
    out2 = _layer2(q0, q1, inv, out1, W_l2p, b_l2, W_r2)
    return (out1, out2)


# speedup vs baseline: 2.3316x; 1.0027x over previous
import functools

import jax
import jax.numpy as jnp
from jax import lax
from jax.experimental import pallas as pl
from jax.experimental.pallas import tpu as pltpu
from jax.experimental.pallas import tpu_sc as plsc

N = 10000
E = 320000
D = 128
NC = 2
NS = 16
NW = NC * NS
CHUNK = 80
EPAD = E
CPW = EPAD // (NW * CHUNK)
NPAD = 10240
RPT = NPAD // NS
CW = 16


def _mesh():
    return plsc.VectorSubcoreMesh(core_axis_name="c", subcore_axis_name="s",
                                  num_cores=NC, num_subcores=NS)


def _sc_sum_body(with_counts, *refs):
    if with_counts:
        (table, packed3, out_p0, out_p1, out_c,
         packed_v, srcl, dstl, rows0, rows1, rows2, frow,
         g0, g1, g2, s0, s1, s2, c0, c1, c2, ones_v, zc, accum, cnt) = refs
        csem = (c0, c1, c2)
    else:
        (table, packed3, out_p0, out_p1,
         packed_v, srcl, dstl, rows0, rows1, rows2, frow,
         g0, g1, g2, s0, s1, s2, accum) = refs
    cid = lax.axis_index("c")
    sid = lax.axis_index("s")
    wid = sid * NC + cid

    rows = (rows0, rows1, rows2)
    gsem = (g0, g1, g2)
    ssem = (s0, s1, s2)

    zero32 = jnp.zeros((32,), jnp.bfloat16)

    def zrow(r, c):
        for c8 in range(D // 32):
            rows0[r, pl.ds(c8 * 32, 32)] = zero32
        return c
    lax.fori_loop(0, CHUNK, zrow, 0)

    def zcopy(k, c):
        pltpu.sync_copy(rows0, accum.at[pl.ds(sid * RPT + k * CHUNK, CHUNK), :])
        return c
    lax.fori_loop(0, RPT // CHUNK, zcopy, 0)

    if with_counts:
        zero16 = jnp.zeros((16,), jnp.float32)
        one16 = jnp.ones((16,), jnp.float32)

        def zcrow(r, c):
            zc[r, :] = zero16
            return c
        lax.fori_loop(0, RPT, zcrow, 0)
        pltpu.sync_copy(zc, cnt.at[pl.ds(sid * RPT, RPT), :])

        def orow(r, c):
            ones_v[r, :] = one16
            return c
        lax.fori_loop(0, CHUNK, orow, 0)

    plsc.subcore_barrier()

    pltpu.sync_copy(packed3.at[wid], packed_v)

    mask14 = jnp.full((16,), 0x3FFF, jnp.int32)

    def unpack(j, b):
        for k in range(CHUNK // 16):
            v = packed_v[j, pl.ds(k * 16, 16)]
            srcl[b, pl.ds(k * 16, 16)] = v & mask14
            dstl[b, pl.ds(k * 16, 16)] = lax.shift_right_logical(
                v, jnp.full((16,), 14, jnp.int32)) & mask14

    def sg(b, j):
        pltpu.async_copy(table.at[srcl.at[b]], rows[b], gsem[b])

    def wg(b):
        pltpu.make_async_copy(table.at[pl.ds(0, CHUNK)], rows[b],
                              gsem[b]).wait()

    def ss(b):
        pltpu.async_copy(rows[b], accum.at[dstl.at[b]], ssem[b], add=True)
        if with_counts:
            pltpu.async_copy(ones_v, cnt.at[dstl.at[b]], csem[b], add=True)

    def ws(b):
        pltpu.make_async_copy(table.at[pl.ds(0, CHUNK)], rows[b],
                              ssem[b]).wait()
        if with_counts:
            pltpu.make_async_copy(out_c.at[0, pl.ds(0, CHUNK), :], ones_v,
                                  csem[b]).wait()

    unpack(0, 0); sg(0, 0)
    unpack(1, 1); sg(1, 1)
    wg(0); ss(0)
    unpack(2, 2); sg(2, 2)
    wg(1); ss(1)
    ws(0); unpack(3, 0); sg(0, 3)
    wg(2); ss(2)
    ws(1); unpack(4, 1); sg(1, 4)

    def tri(t, c):
        j0 = 3 * t
        wg(0); ss(0)
        ws(2); unpack(j0 + 2, 2); sg(2, j0 + 2)
        wg(1); ss(1)
        ws(0); unpack(j0 + 3, 0); sg(0, j0 + 3)
        wg(2); ss(2)
        ws(1); unpack(j0 + 4, 1); sg(1, j0 + 4)
        return c
    lax.fori_loop(1, (CPW - 5) // 3 + 1, tri, 0)

    wg(0); ss(0)
    wg(1); ws(2); ss(1)
    ws(0); ws(1)

    plsc.subcore_barrier()

    sh16 = jnp.full((16,), 16, jnp.int32)
    himask = jnp.full((16,), -65536, jnp.int32)

    def conv_chunk(k, c):
        base = sid * RPT + k * CHUNK
        pltpu.sync_copy(accum.at[pl.ds(base, CHUNK), :], rows0)

        def conv_row(r, c2):
            for g in range(D // 32):
                w = plsc.bitcast(rows0[r, pl.ds(32 * g, 32)], jnp.int32)
                ev = plsc.bitcast(lax.shift_left(w, sh16), jnp.float32)
                od = plsc.bitcast(w & himask, jnp.float32)
                frow[r, pl.ds(16 * g, 16)] = ev
                frow[r, pl.ds(D // 2 + 16 * g, 16)] = od
            return c2
        lax.fori_loop(0, CHUNK, conv_row, 0)

        @pl.when(cid == 0)
        def _():
            pltpu.sync_copy(frow, out_p0.at[pl.ds(base, CHUNK), :])

        @pl.when(cid == 1)
        def _():
            pltpu.sync_copy(frow, out_p1.at[pl.ds(base, CHUNK), :])
        return c
    lax.fori_loop(0, RPT // CHUNK, conv_chunk, 0)

    if with_counts:
        pltpu.sync_copy(cnt.at[pl.ds(sid * RPT, RPT), :],
                        out_c.at[cid, pl.ds(sid * RPT, RPT), :])


def _make_sc_sum(with_counts):
    out_type = [jax.ShapeDtypeStruct((NPAD, D), jnp.float32),
                jax.ShapeDtypeStruct((NPAD, D), jnp.float32)]
    scratch = [
        pltpu.VMEM((CPW, CHUNK), jnp.int32),
        pltpu.VMEM((3, CHUNK), jnp.int32),
        pltpu.VMEM((3, CHUNK), jnp.int32),
        pltpu.VMEM((CHUNK, D), jnp.bfloat16),
        pltpu.VMEM((CHUNK, D), jnp.bfloat16),
        pltpu.VMEM((CHUNK, D), jnp.bfloat16),
        pltpu.VMEM((CHUNK, D), jnp.float32),
        pltpu.SemaphoreType.DMA,
        pltpu.SemaphoreType.DMA,
        pltpu.SemaphoreType.DMA,
        pltpu.SemaphoreType.DMA,
        pltpu.SemaphoreType.DMA,
        pltpu.SemaphoreType.DMA,
    ]
    if with_counts:
        out_type.append(jax.ShapeDtypeStruct((NC, NPAD, CW), jnp.float32))
        scratch += [
            pltpu.SemaphoreType.DMA,
            pltpu.SemaphoreType.DMA,
            pltpu.SemaphoreType.DMA,
            pltpu.VMEM((CHUNK, CW), jnp.float32),
            pltpu.VMEM((RPT, CW), jnp.float32),
        ]
    scratch.append(pltpu.VMEM_SHARED((NPAD, D), jnp.bfloat16))
    if with_counts:
        scratch.append(pltpu.VMEM_SHARED((NPAD, CW), jnp.float32))
    return pl.kernel(
        functools.partial(_sc_sum_body, with_counts),
        out_type=tuple(out_type),
        mesh=_mesh(),
        scratch_types=tuple(scratch),
        compiler_params=pltpu.CompilerParams(use_tc_tiling_on_sc=False,
                                             needs_layout_passes=False),
        name="sc_segment_sum" + ("_counts" if with_counts else ""),
    )


_BN = 1000


def _pack_body(e_ref, pk_ref):
    pk_ref[...] = e_ref[0] | (e_ref[1] << 14)


def _pack(e):
    return pl.pallas_call(
        _pack_body,
        out_shape=jax.ShapeDtypeStruct((EPAD,), jnp.int32),
    )(e)


def _proj_body(x_ref, w_ref, b_ref, o_ref, ob_ref):
    t = lax.dot_general(
        x_ref[...], w_ref[...], (((1,), (1,)), ((), ())),
        preferred_element_type=jnp.float32) + b_ref[...]
    o_ref[...] = t
    ob_ref[...] = t.astype(jnp.bfloat16).reshape(_BN * D)


def _proj(x, w, b):
    return pl.pallas_call(
        _proj_body,
        grid=(N // _BN,),
        in_specs=[
            pl.BlockSpec((_BN, D), lambda i: (i, 0)),
            pl.BlockSpec((D, D), lambda i: (0, 0)),
            pl.BlockSpec((1, D), lambda i: (0, 0)),
        ],
        out_specs=[pl.BlockSpec((_BN, D), lambda i: (i, 0)),
                   pl.BlockSpec((_BN * D,), lambda i: (i,))],
        out_shape=[jax.ShapeDtypeStruct((N, D), jnp.float32),
                   jax.ShapeDtypeStruct((N * D,), jnp.bfloat16)],
    )(x, w, b.reshape(1, D))


def _layer1_body(p0_ref, p1_ref, c_ref, h_ref, wl_ref, bl_ref, wr_ref,
                 o_ref, ob_ref, inv_ref):
    summed = p0_ref[...] + p1_ref[...]
    inv = 1.0 / jnp.maximum(c_ref[0] + c_ref[1], 1.0)
    inv_ref[...] = inv
    t = (lax.dot_general(summed * inv[:, 0:1], wl_ref[...],
                         (((1,), (1,)), ((), ())),
                         preferred_element_type=jnp.float32)
         + bl_ref[...]
         + lax.dot_general(h_ref[...], wr_ref[...], (((1,), (1,)), ((), ())),
                           preferred_element_type=jnp.float32))
    t = jnp.maximum(t, 0.0) + h_ref[...]
    o_ref[...] = t
    ob_ref[...] = t.astype(jnp.bfloat16).reshape(_BN * D)


def _layer1(p0, p1, c, h, wl, bl, wr):
    return pl.pallas_call(
        _layer1_body,
        grid=(N // _BN,),
        in_specs=[
            pl.BlockSpec((_BN, D), lambda i: (i, 0)),
            pl.BlockSpec((_BN, D), lambda i: (i, 0)),
            pl.BlockSpec((NC, _BN, CW), lambda i: (0, i, 0)),
            pl.BlockSpec((_BN, D), lambda i: (i, 0)),
            pl.BlockSpec((D, D), lambda i: (0, 0)),
            pl.BlockSpec((1, D), lambda i: (0, 0)),
            pl.BlockSpec((D, D), lambda i: (0, 0)),
        ],
        out_specs=[pl.BlockSpec((_BN, D), lambda i: (i, 0)),
                   pl.BlockSpec((_BN * D,), lambda i: (i,)),
                   pl.BlockSpec((_BN, CW), lambda i: (i, 0))],
        out_shape=[jax.ShapeDtypeStruct((N, D), jnp.float32),
                   jax.ShapeDtypeStruct((N * D,), jnp.bfloat16),
                   jax.ShapeDtypeStruct((N, CW), jnp.float32)],
    )(p0, p1, c, h, wl, bl.reshape(1, D), wr)


def _layer2_body(p0_ref, p1_ref, inv_ref, h_ref, wl_ref, bl_ref, wr_ref,
                 o_ref):
    mean = (p0_ref[...] + p1_ref[...]) * inv_ref[:, 0:1]
    t = (lax.dot_general(mean, wl_ref[...], (((1,), (1,)), ((), ())),
                         preferred_element_type=jnp.float32)
         + bl_ref[...]
         + lax.dot_general(h_ref[...], wr_ref[...], (((1,), (1,)), ((), ())),
                           preferred_element_type=jnp.float32))
    o_ref[...] = t + h_ref[...]


def _layer2(p0, p1, inv, h, wl, bl, wr):
    return pl.pallas_call(
        _layer2_body,
        grid=(N // _BN,),
        in_specs=[
            pl.BlockSpec((_BN, D), lambda i: (i, 0)),
            pl.BlockSpec((_BN, D), lambda i: (i, 0)),
            pl.BlockSpec((_BN, CW), lambda i: (i, 0)),
            pl.BlockSpec((_BN, D), lambda i: (i, 0)),
            pl.BlockSpec((D, D), lambda i: (0, 0)),
            pl.BlockSpec((1, D), lambda i: (0, 0)),
            pl.BlockSpec((D, D), lambda i: (0, 0)),
        ],
        out_specs=pl.BlockSpec((_BN, D), lambda i: (i, 0)),
        out_shape=jax.ShapeDtypeStruct((N, D), jnp.float32),
    )(p0, p1, inv, h, wl, bl.reshape(1, D), wr)


_sc_sum_counts = _make_sc_sum(True)
_sc_sum_plain = _make_sc_sum(False)


def kernel(x, edge_index, W_lin, b_lin, W_l1, b_l1, W_r1, W_l2, b_l2, W_r2):
    perm = jnp.concatenate([jnp.arange(0, D, 2), jnp.arange(1, D, 2)])
    W_l1p = W_l1[:, perm]
    W_l2p = W_l2[:, perm]
    packed3 = _pack(edge_index).reshape(NW, CPW, CHUNK)
    h, h_bf = _proj(x, W_lin, b_lin)
    p0, p1, c = _sc_sum_counts(h_bf.reshape(N, D), packed3)
    out1, out1_bf, inv = _layer1(p0, p1, c, h, W_l1p, b_l1, W_r1)
    q0, q1 = _sc_sum_plain(out1_bf.reshape(N, D), packed3)
    out2 = _layer2(q0, q1, inv, out1, W_l2p, b_l2, W_r2)
    return (out1, out2)
